# baseline (speedup 1.0000x reference)
import functools

import jax
import jax.numpy as jnp
import numpy as np
from jax import lax
from jax.experimental import pallas as pl
from jax.experimental.pallas import tpu as pltpu
from jax.experimental.pallas import tpu_sc as plsc

_LEVELS = 5
_D = 8
_NBINS = _LEVELS ** _D
_HPAD = 390656

_ROWS = 2048
_COLS = 1024
_BR = 256

_N_IDX = _ROWS * _COLS // _D
_IDX_COLS = 128

_NUM_TILES = 16
_TILE_BINS = _HPAD // _NUM_TILES
_TILE_IDX_ROWS = _ROWS // _NUM_TILES


def _selection_matrix():
    s = np.zeros((_COLS, 2 * _IDX_COLS), dtype=np.float32)
    for c in range(_COLS):
        g, k = divmod(c, _D)
        if k < 4:
            s[c, g] = 5.0 ** k
        else:
            s[c, _IDX_COLS + g] = 5.0 ** (k - 4)
    return jnp.asarray(s, dtype=jnp.bfloat16)


def _tc_body(x_ref, s_ref, q_ref, zhat_ref, idx_ref):
    x = x_ref[...]
    z2 = jnp.tanh(x) * 2.0
    zhat = jnp.round(z2)
    q_ref[...] = zhat * 0.5
    zhat_ref[...] = zhat
    zb = (zhat + 2.0).astype(jnp.bfloat16)
    p = jnp.dot(zb, s_ref[...], preferred_element_type=jnp.float32)
    idx_ref[...] = (p[:, :_IDX_COLS] + 625.0 * p[:, _IDX_COLS:]).astype(
        jnp.int32)


def _quantize_tc(x2d, s):
    return pl.pallas_call(
        _tc_body,
        grid=(_ROWS // _BR,),
        in_specs=[
            pl.BlockSpec((_BR, _COLS), lambda i: (i, 0)),
            pl.BlockSpec((_COLS, 2 * _IDX_COLS), lambda i: (0, 0)),
        ],
        out_specs=[
            pl.BlockSpec((_BR, _COLS), lambda i: (i, 0)),
            pl.BlockSpec((_BR, _COLS), lambda i: (i, 0)),
            pl.BlockSpec((_BR, _IDX_COLS), lambda i: (i, 0)),
        ],
        out_shape=[
            jax.ShapeDtypeStruct((_ROWS, _COLS), jnp.float32),
            jax.ShapeDtypeStruct((_ROWS, _COLS), jnp.float32),
            jax.ShapeDtypeStruct((_ROWS, _IDX_COLS), jnp.int32),
        ],
    )(x2d, s)


def _hist_body(idx_hbm, out_hbm, idx_v, ones_v, fill_v, hist_sh, sem):
    core = lax.axis_index("c")
    sid = lax.axis_index("s")

    @pl.when(core == 0)
    def _():
        @pl.loop(0, _TILE_BINS, step=16)
        def _(i):
            fill_v[pl.ds(i, 16)] = jnp.zeros((16,), jnp.int32)

        pltpu.sync_copy(fill_v, hist_sh.at[pl.ds(sid * _TILE_BINS, _TILE_BINS)])

        @pl.loop(0, _IDX_COLS, step=16)
        def _(i):
            ones_v[pl.ds(i, 16)] = jnp.ones((16,), jnp.int32)

        pltpu.sync_copy(
            idx_hbm.at[pl.ds(sid * _TILE_IDX_ROWS, _TILE_IDX_ROWS)], idx_v)

        plsc.subcore_barrier()

        @pl.loop(0, _TILE_IDX_ROWS)
        def _(j):
            pltpu.sync_copy(ones_v, hist_sh.at[idx_v.at[j]], add=True)

        plsc.subcore_barrier()

        pltpu.sync_copy(
            hist_sh.at[pl.ds(sid * _TILE_BINS, _TILE_BINS)], fill_v)
        pltpu.sync_copy(
            fill_v, out_hbm.at[pl.ds(sid * _TILE_BINS, _TILE_BINS)])


def _usage_sc(idx):
    mesh = plsc.VectorSubcoreMesh(core_axis_name="c", subcore_axis_name="s")
    k = pl.kernel(
        _hist_body,
        out_type=jax.ShapeDtypeStruct((_HPAD,), jnp.int32),
        mesh=mesh,
        scratch_types=[
            pltpu.VMEM((_TILE_IDX_ROWS, _IDX_COLS), jnp.int32),
            pltpu.VMEM((_IDX_COLS,), jnp.int32),
            pltpu.VMEM((_TILE_BINS,), jnp.int32),
            pltpu.VMEM_SHARED((_HPAD,), jnp.int32),
            pltpu.SemaphoreType.DMA,
        ],
    )
    return k(idx)


def kernel(x):
    x2d = x.reshape(_ROWS, _COLS)
    s = _selection_matrix()
    quantized, zhat, idx = _quantize_tc(x2d, s)
    usage = _usage_sc(idx)[:_NBINS]
    return (
        quantized.reshape(x.shape),
        zhat.reshape(x.shape),
        usage,
    )
